# Initial kernel scaffold; baseline (speedup 1.0000x reference)
import jax
import jax.numpy as jnp
from jax.experimental import pallas as pl


def kernel(h, edge_index, ws_embed, W_fc, W_feat, b_feat, W_attn):
    raise NotImplementedError("write your pallas kernel here")


# speedup vs baseline: 11.1304x; 11.1304x over previous
import functools

import jax
import jax.numpy as jnp
from jax import lax
from jax.experimental import pallas as pl
from jax.experimental.pallas import tpu as pltpu
from jax.experimental.pallas import tpu_sc as plsc

N = 10000
E = 320000
D = 128
DH = D // 2

NC = 2
NS = 16
EPT = E // NS
G = 80
NG = EPT // G

DEN_CHUNK = 632
DEN_LAST = N - 15 * DEN_CHUNK


def _tc_zs_body(h_ref, wfc_ref, wab_ref, z_ref, s_ref):
    z = jnp.dot(h_ref[...], wfc_ref[...], preferred_element_type=jnp.float32)
    z_ref[0] = z[:, :DH]
    z_ref[1] = z[:, DH:]
    s_ref[...] = jnp.dot(z, wab_ref[...], preferred_element_type=jnp.float32)


def _tc_zs(h, W_fc, Wab):
    R = 2000
    return pl.pallas_call(
        _tc_zs_body,
        grid=(N // R,),
        in_specs=[
            pl.BlockSpec((R, D), lambda i: (i, 0)),
            pl.BlockSpec((D, D), lambda i: (0, 0)),
            pl.BlockSpec((D, 2), lambda i: (0, 0)),
        ],
        out_specs=[
            pl.BlockSpec((NC, R, DH), lambda i: (0, i, 0)),
            pl.BlockSpec((R, 2), lambda i: (i, 0)),
        ],
        out_shape=[
            jax.ShapeDtypeStruct((NC, N, DH), jnp.float32),
            jax.ShapeDtypeStruct((N, 2), jnp.float32),
        ],
    )(h, W_fc, Wab)


def _sc_body(src_hbm, dst_hbm, sa_hbm, sb_hbm, z_hbm,
             outp_hbm, denp_hbm,
             src_v, dst_v, sa_v, sb_v, exv, zbuf, zerov, dzv,
             out_sh, den_sh, sem):
    c = lax.axis_index("c")
    s = lax.axis_index("s")

    pltpu.sync_copy(src_hbm.at[s], src_v)
    pltpu.sync_copy(dst_hbm.at[s], dst_v)
    pltpu.sync_copy(sa_hbm, sa_v)
    pltpu.sync_copy(sb_hbm, sb_v)

    zv16 = jnp.zeros((16,), jnp.float32)

    def _zero_row(r, carry):
        for k in range(DH // 16):
            zerov[r, pl.ds(k * 16, 16)] = zv16
        return carry

    lax.fori_loop(0, 160, _zero_row, 0)
    for k in range(40):
        dzv[pl.ds(k * 16, 16)] = zv16

    @pl.when(s < 15)
    def _():
        base = s * DEN_CHUNK
        for off, sz in ((0, 160), (160, 160), (320, 160), (480, 152)):
            pltpu.sync_copy(zerov.at[pl.ds(0, sz)],
                            out_sh.at[pl.ds(base + off, sz)])
        pltpu.sync_copy(dzv.at[pl.ds(0, DEN_CHUNK)],
                        den_sh.at[pl.ds(base, DEN_CHUNK)])

    @pl.when(s == 15)
    def _():
        base = 15 * DEN_CHUNK
        for off, sz in ((0, 160), (160, 160), (320, 160), (480, 40)):
            pltpu.sync_copy(zerov.at[pl.ds(0, sz)],
                            out_sh.at[pl.ds(base + off, sz)])
        pltpu.sync_copy(dzv.at[pl.ds(0, DEN_LAST)],
                        den_sh.at[pl.ds(base, DEN_LAST)])

    plsc.subcore_barrier()

    def _group(g, carry):
        for k in range(G // 16):
            si = src_v[g, pl.ds(k * 16, 16)]
            di = dst_v[g, pl.ds(k * 16, 16)]
            av = plsc.load_gather(sa_v, [si])
            bv = plsc.load_gather(sb_v, [di])
            x = av + bv
            ex = jnp.exp(jnp.where(x > 0, x, x * 0.01))
            exv[pl.ds(k * 16, 16)] = ex

        pltpu.async_copy(z_hbm.at[c].at[src_v.at[g]], zbuf, sem).wait()

        def _scale16(t, cc):
            avec = exv[pl.ds(t * 16, 16)]
            for l in range(16):
                a = avec[l]
                j = t * 16 + l
                for k in range(DH // 16):
                    sl = pl.ds(k * 16, 16)
                    zbuf[j, sl] = zbuf[j, sl] * a
            return cc

        lax.fori_loop(0, G // 16, _scale16, 0)

        pltpu.sync_copy(zbuf, out_sh.at[dst_v.at[g]], add=True)
        pltpu.sync_copy(exv, den_sh.at[dst_v.at[g]], add=True)
        return carry

    lax.fori_loop(0, NG, _group, 0)

    plsc.subcore_barrier()

    @pl.when(s < 15)
    def _():
        base = s * DEN_CHUNK
        pltpu.sync_copy(out_sh.at[pl.ds(base, DEN_CHUNK)],
                        outp_hbm.at[c, pl.ds(base, DEN_CHUNK)])

        @pl.when(c == 0)
        def _():
            pltpu.sync_copy(den_sh.at[pl.ds(base, DEN_CHUNK)],
                            dzv.at[pl.ds(0, DEN_CHUNK)])
            pltpu.sync_copy(dzv.at[pl.ds(0, DEN_CHUNK)],
                            denp_hbm.at[pl.ds(base, DEN_CHUNK)])

    @pl.when(s == 15)
    def _():
        base = 15 * DEN_CHUNK
        pltpu.sync_copy(out_sh.at[pl.ds(base, DEN_LAST)],
                        outp_hbm.at[c, pl.ds(base, DEN_LAST)])

        @pl.when(c == 0)
        def _():
            pltpu.sync_copy(den_sh.at[pl.ds(base, DEN_LAST)],
                            dzv.at[pl.ds(0, DEN_LAST)])
            pltpu.sync_copy(dzv.at[pl.ds(0, DEN_LAST)],
                            denp_hbm.at[pl.ds(base, DEN_LAST)])


_sc_kernel = functools.partial(
    pl.kernel,
    out_type=[
        jax.ShapeDtypeStruct((NC, N, DH), jnp.float32),
        jax.ShapeDtypeStruct((N,), jnp.float32),
    ],
    mesh=plsc.VectorSubcoreMesh(core_axis_name="c", subcore_axis_name="s"),
    compiler_params=pltpu.CompilerParams(needs_layout_passes=False,
                                         use_tc_tiling_on_sc=False),
    scratch_types=[
        pltpu.VMEM((NG, G), jnp.int32),
        pltpu.VMEM((NG, G), jnp.int32),
        pltpu.VMEM((N,), jnp.float32),
        pltpu.VMEM((N,), jnp.float32),
        pltpu.VMEM((G,), jnp.float32),
        pltpu.VMEM((G, DH), jnp.float32),
        pltpu.VMEM((160, DH), jnp.float32),
        pltpu.VMEM((640,), jnp.float32),
        pltpu.VMEM_SHARED((N, DH), jnp.float32),
        pltpu.VMEM_SHARED((N,), jnp.float32),
        pltpu.SemaphoreType.DMA,
    ],
)(_sc_body)


def _tc_norm_body(p_ref, d_ref, o_ref):
    num = jnp.concatenate([p_ref[0], p_ref[1]], axis=1)
    o_ref[...] = num / jnp.maximum(d_ref[...], 1e-16)


def _tc_norm(outp, denp):
    R = 2000
    return pl.pallas_call(
        _tc_norm_body,
        grid=(N // R,),
        in_specs=[
            pl.BlockSpec((NC, R, DH), lambda i: (0, i, 0)),
            pl.BlockSpec((R, 1), lambda i: (i, 0)),
        ],
        out_specs=pl.BlockSpec((R, D), lambda i: (i, 0)),
        out_shape=jax.ShapeDtypeStruct((N, D), jnp.float32),
    )(outp, denp)


def kernel(h, edge_index, ws_embed, W_fc, W_feat, b_feat, W_attn):
    src = edge_index[0].astype(jnp.int32).reshape(NS, NG, G)
    dst = edge_index[1].astype(jnp.int32).reshape(NS, NG, G)
    Wab = jnp.concatenate([W_attn[:D], W_attn[D:]], axis=1)
    z2, s2 = _tc_zs(h, W_fc, Wab)
    sa = s2[:, 0]
    sb = s2[:, 1]
    outp, denp = _sc_kernel(src, dst, sa, sb, z2)
    return _tc_norm(outp, denp.reshape(N, 1))


# speedup vs baseline: 14.3512x; 1.2894x over previous
import functools

import jax
import jax.numpy as jnp
from jax import lax
from jax.experimental import pallas as pl
from jax.experimental.pallas import tpu as pltpu
from jax.experimental.pallas import tpu_sc as plsc

N = 10000
E = 320000
D = 128
DH = D // 2

NC = 2
NS = 16
EPT = E // NS
G = 80
NG = EPT // G
UN = 2

CHUNK = 632
LAST = N - 15 * CHUNK


def _tc_zs_body(h_ref, wfc_ref, wab_ref, z_ref, s_ref):
    z = jnp.dot(h_ref[...], wfc_ref[...], preferred_element_type=jnp.float32)
    z_ref[0] = z[:, :DH]
    z_ref[1] = z[:, DH:]
    s_ref[...] = jnp.dot(z, wab_ref[...], preferred_element_type=jnp.float32)


def _tc_zs(h, W_fc, Wab):
    R = 2000
    return pl.pallas_call(
        _tc_zs_body,
        grid=(N // R,),
        in_specs=[
            pl.BlockSpec((R, D), lambda i: (i, 0)),
            pl.BlockSpec((D, D), lambda i: (0, 0)),
            pl.BlockSpec((D, 2), lambda i: (0, 0)),
        ],
        out_specs=[
            pl.BlockSpec((NC, R, DH), lambda i: (0, i, 0)),
            pl.BlockSpec((R, 2), lambda i: (i, 0)),
        ],
        out_shape=[
            jax.ShapeDtypeStruct((NC, N, DH), jnp.float32),
            jax.ShapeDtypeStruct((N, 2), jnp.float32),
        ],
    )(h, W_fc, Wab)


def _sc_body(src_hbm, dst_hbm, sa_hbm, sb_hbm, z_hbm,
             outp_hbm, denp_hbm,
             src_v, dst_v, sa_v, sb_v,
             b0, b1, e0, e1, zerov, dzv,
             out_sh, den_sh,
             g0, g1, w0, w1, d0, d1):
    c = lax.axis_index("c")
    s = lax.axis_index("s")
    bufs = (b0, b1)
    exvs = (e0, e1)
    gsems = (g0, g1)
    wsems = (w0, w1)
    dsems = (d0, d1)

    pltpu.sync_copy(src_hbm.at[s], src_v)
    pltpu.sync_copy(dst_hbm.at[s], dst_v)
    pltpu.sync_copy(sa_hbm, sa_v)
    pltpu.sync_copy(sb_hbm, sb_v)

    zv16 = jnp.zeros((16,), jnp.float32)

    def _zero_row(r, carry):
        for k in range(DH // 16):
            zerov[r, pl.ds(k * 16, 16)] = zv16
        return carry

    lax.fori_loop(0, 160, _zero_row, 0)
    for k in range(40):
        dzv[pl.ds(k * 16, 16)] = zv16

    @pl.when(s < 15)
    def _():
        base = s * CHUNK
        for off, sz in ((0, 160), (160, 160), (320, 160), (480, 152)):
            pltpu.sync_copy(zerov.at[pl.ds(0, sz)],
                            out_sh.at[pl.ds(base + off, sz)])
        pltpu.sync_copy(dzv.at[pl.ds(0, CHUNK)],
                        den_sh.at[pl.ds(base, CHUNK)])

    @pl.when(s == 15)
    def _():
        base = 15 * CHUNK
        for off, sz in ((0, 160), (160, 160), (320, 160), (480, 40)):
            pltpu.sync_copy(zerov.at[pl.ds(0, sz)],
                            out_sh.at[pl.ds(base + off, sz)])
        pltpu.sync_copy(dzv.at[pl.ds(0, LAST)],
                        den_sh.at[pl.ds(base, LAST)])

    plsc.subcore_barrier()

    def _body(t, carry):
        gdescs = []
        for i in range(UN):
            g = t * UN + i
            gdescs.append(pltpu.async_copy(
                z_hbm.at[c].at[src_v.at[g]], bufs[i], gsems[i]))

        for i in range(UN):
            g = t * UN + i
            for k in range(G // 16):
                si = src_v[g, pl.ds(k * 16, 16)]
                di = dst_v[g, pl.ds(k * 16, 16)]
                av = plsc.load_gather(sa_v, [si])
                bv = plsc.load_gather(sb_v, [di])
                x = av + bv
                exvs[i][pl.ds(k * 16, 16)] = jnp.exp(
                    jnp.where(x > 0, x, x * 0.01))

        wdescs = []
        for i in range(UN):
            g = t * UN + i
            gdescs[i].wait()
            buf = bufs[i]
            exv = exvs[i]

            def _scale16(tt, cc, buf=buf, exv=exv):
                avec = exv[pl.ds(tt * 16, 16)]
                for l in range(16):
                    a = avec[l]
                    j = tt * 16 + l
                    for k in range(DH // 16):
                        sl = pl.ds(k * 16, 16)
                        buf[j, sl] = buf[j, sl] * a
                return cc

            lax.fori_loop(0, G // 16, _scale16, 0)
            wdescs.append(pltpu.async_copy(
                buf, out_sh.at[dst_v.at[g]], wsems[i], add=True))
            wdescs.append(pltpu.async_copy(
                exv, den_sh.at[dst_v.at[g]], dsems[i], add=True))

        for d in wdescs:
            d.wait()
        return carry

    lax.fori_loop(0, NG // UN, _body, 0)

    plsc.subcore_barrier()

    @pl.when(s < 15)
    def _():
        base = s * CHUNK
        pltpu.sync_copy(out_sh.at[pl.ds(base, CHUNK)],
                        outp_hbm.at[c, pl.ds(base, CHUNK)])

        @pl.when(c == 0)
        def _():
            pltpu.sync_copy(den_sh.at[pl.ds(base, CHUNK)],
                            dzv.at[pl.ds(0, CHUNK)])
            pltpu.sync_copy(dzv.at[pl.ds(0, CHUNK)],
                            denp_hbm.at[pl.ds(base, CHUNK)])

    @pl.when(s == 15)
    def _():
        base = 15 * CHUNK
        pltpu.sync_copy(out_sh.at[pl.ds(base, LAST)],
                        outp_hbm.at[c, pl.ds(base, LAST)])

        @pl.when(c == 0)
        def _():
            pltpu.sync_copy(den_sh.at[pl.ds(base, LAST)],
                            dzv.at[pl.ds(0, LAST)])
            pltpu.sync_copy(dzv.at[pl.ds(0, LAST)],
                            denp_hbm.at[pl.ds(base, LAST)])


_sc_kernel = functools.partial(
    pl.kernel,
    out_type=[
        jax.ShapeDtypeStruct((NC, N, DH), jnp.float32),
        jax.ShapeDtypeStruct((N,), jnp.float32),
    ],
    mesh=plsc.VectorSubcoreMesh(core_axis_name="c", subcore_axis_name="s"),
    compiler_params=pltpu.CompilerParams(needs_layout_passes=False,
                                         use_tc_tiling_on_sc=False),
    scratch_types=(
        [
            pltpu.VMEM((NG, G), jnp.int32),
            pltpu.VMEM((NG, G), jnp.int32),
            pltpu.VMEM((N,), jnp.float32),
            pltpu.VMEM((N,), jnp.float32),
        ]
        + [pltpu.VMEM((G, DH), jnp.float32) for _ in range(UN)]
        + [pltpu.VMEM((G,), jnp.float32) for _ in range(UN)]
        + [
            pltpu.VMEM((160, DH), jnp.float32),
            pltpu.VMEM((640,), jnp.float32),
            pltpu.VMEM_SHARED((N, DH), jnp.float32),
            pltpu.VMEM_SHARED((N,), jnp.float32),
        ]
        + [pltpu.SemaphoreType.DMA for _ in range(3 * UN)]
    ),
)(_sc_body)


def _tc_norm_body(p_ref, d_ref, o_ref):
    num = jnp.concatenate([p_ref[0], p_ref[1]], axis=1)
    o_ref[...] = num / jnp.maximum(d_ref[...], 1e-16)


def _tc_norm(outp, denp):
    R = 2000
    return pl.pallas_call(
        _tc_norm_body,
        grid=(N // R,),
        in_specs=[
            pl.BlockSpec((NC, R, DH), lambda i: (0, i, 0)),
            pl.BlockSpec((R, 1), lambda i: (i, 0)),
        ],
        out_specs=pl.BlockSpec((R, D), lambda i: (i, 0)),
        out_shape=jax.ShapeDtypeStruct((N, D), jnp.float32),
    )(outp, denp)


def kernel(h, edge_index, ws_embed, W_fc, W_feat, b_feat, W_attn):
    src = edge_index[0].astype(jnp.int32).reshape(NS, NG, G)
    dst = edge_index[1].astype(jnp.int32).reshape(NS, NG, G)
    Wab = jnp.concatenate([W_attn[:D], W_attn[D:]], axis=1)
    z2, s2 = _tc_zs(h, W_fc, Wab)
    sa = s2[:, 0]
    sb = s2[:, 1]
    outp, denp = _sc_kernel(src, dst, sa, sb, z2)
    return _tc_norm(outp, denp.reshape(N, 1))


# speedup vs baseline: 27.1054x; 1.8887x over previous
import functools

import jax
import jax.numpy as jnp
from jax import lax
from jax.experimental import pallas as pl
from jax.experimental.pallas import tpu as pltpu
from jax.experimental.pallas import tpu_sc as plsc

N = 10000
E = 320000
D = 128
DH = D // 2

NC = 2
NS = 16
EPT = E // NS
G = 80
NG = EPT // G
UN = 2

CHUNK = 632
LAST = N - 15 * CHUNK


def _tc_zs_body(h_ref, wfc_ref, wab_ref, z_ref, s_ref):
    z = jnp.dot(h_ref[...], wfc_ref[...], preferred_element_type=jnp.float32)
    z_ref[0] = z[:, :DH]
    z_ref[1] = z[:, DH:]
    s_ref[...] = jnp.dot(z, wab_ref[...], preferred_element_type=jnp.float32)


def _tc_zs(h, W_fc, Wab):
    R = 2000
    return pl.pallas_call(
        _tc_zs_body,
        grid=(N // R,),
        in_specs=[
            pl.BlockSpec((R, D), lambda i: (i, 0)),
            pl.BlockSpec((D, D), lambda i: (0, 0)),
            pl.BlockSpec((D, 2), lambda i: (0, 0)),
        ],
        out_specs=[
            pl.BlockSpec((NC, R, DH), lambda i: (0, i, 0)),
            pl.BlockSpec((R, 2), lambda i: (i, 0)),
        ],
        out_shape=[
            jax.ShapeDtypeStruct((NC, N, DH), jnp.float32),
            jax.ShapeDtypeStruct((N, 2), jnp.float32),
        ],
    )(h, W_fc, Wab)


def _sc_body(src_hbm, dst_hbm, sa_hbm, sb_hbm, z_hbm,
             outp_hbm, denp_hbm,
             src_v, dst_v, sa_v, sb_v,
             b0, b1, e0, e1, zerov, dzv,
             out_sh, den_sh,
             g0, g1, wsem, dsem):
    c = lax.axis_index("c")
    s = lax.axis_index("s")
    bufs = (b0, b1)
    exvs = (e0, e1)
    gsems = (g0, g1)

    pltpu.sync_copy(src_hbm.at[s], src_v)
    pltpu.sync_copy(dst_hbm.at[s], dst_v)
    pltpu.sync_copy(sa_hbm, sa_v)
    pltpu.sync_copy(sb_hbm, sb_v)

    zv16 = jnp.zeros((16,), jnp.float32)

    def _zero_row(r, carry):
        for k in range(DH // 16):
            zerov[r, pl.ds(k * 16, 16)] = zv16
        return carry

    lax.fori_loop(0, 160, _zero_row, 0)
    for k in range(40):
        dzv[pl.ds(k * 16, 16)] = zv16

    @pl.when(s < 15)
    def _():
        base = s * CHUNK
        for off, sz in ((0, 160), (160, 160), (320, 160), (480, 152)):
            pltpu.sync_copy(zerov.at[pl.ds(0, sz)],
                            out_sh.at[pl.ds(base + off, sz)])
        pltpu.sync_copy(dzv.at[pl.ds(0, CHUNK)],
                        den_sh.at[pl.ds(base, CHUNK)])

    @pl.when(s == 15)
    def _():
        base = 15 * CHUNK
        for off, sz in ((0, 160), (160, 160), (320, 160), (480, 40)):
            pltpu.sync_copy(zerov.at[pl.ds(0, sz)],
                            out_sh.at[pl.ds(base + off, sz)])
        pltpu.sync_copy(dzv.at[pl.ds(0, LAST)],
                        den_sh.at[pl.ds(base, LAST)])

    plsc.subcore_barrier()

    def _body(t, carry):
        gdescs = []
        for i in range(UN):
            g = t * UN + i
            gdescs.append(pltpu.async_copy(
                z_hbm.at[c].at[src_v.at[g]], bufs[i], gsems[i]))

        for i in range(UN):
            g = t * UN + i
            for k in range(G // 16):
                si = src_v[g, pl.ds(k * 16, 16)]
                di = dst_v[g, pl.ds(k * 16, 16)]
                av = plsc.load_gather(sa_v, [si])
                bv = plsc.load_gather(sb_v, [di])
                x = av + bv
                exvs[i][pl.ds(k * 16, 16)] = jnp.exp(
                    jnp.where(x > 0, x, x * 0.01))

        wdescs = []
        for i in range(UN):
            g = t * UN + i
            gdescs[i].wait()
            buf = bufs[i]
            exv = exvs[i]
            for tt in range(G // 16):
                avec = exv[pl.ds(tt * 16, 16)]
                for l in range(16):
                    a = avec[l]
                    j = tt * 16 + l
                    for k in range(DH // 16):
                        sl = pl.ds(k * 16, 16)
                        buf[j, sl] = buf[j, sl] * a
            wdescs.append(pltpu.async_copy(
                buf, out_sh.at[dst_v.at[g]], wsem, add=True))
            wdescs.append(pltpu.async_copy(
                exv, den_sh.at[dst_v.at[g]], dsem, add=True))

        for d in wdescs:
            d.wait()
        return carry

    lax.fori_loop(0, NG // UN, _body, 0)

    plsc.subcore_barrier()

    @pl.when(s < 15)
    def _():
        base = s * CHUNK
        pltpu.sync_copy(out_sh.at[pl.ds(base, CHUNK)],
                        outp_hbm.at[c, pl.ds(base, CHUNK)])

        @pl.when(c == 0)
        def _():
            pltpu.sync_copy(den_sh.at[pl.ds(base, CHUNK)],
                            dzv.at[pl.ds(0, CHUNK)])
            pltpu.sync_copy(dzv.at[pl.ds(0, CHUNK)],
                            denp_hbm.at[pl.ds(base, CHUNK)])

    @pl.when(s == 15)
    def _():
        base = 15 * CHUNK
        pltpu.sync_copy(out_sh.at[pl.ds(base, LAST)],
                        outp_hbm.at[c, pl.ds(base, LAST)])

        @pl.when(c == 0)
        def _():
            pltpu.sync_copy(den_sh.at[pl.ds(base, LAST)],
                            dzv.at[pl.ds(0, LAST)])
            pltpu.sync_copy(dzv.at[pl.ds(0, LAST)],
                            denp_hbm.at[pl.ds(base, LAST)])


_sc_kernel = functools.partial(
    pl.kernel,
    out_type=[
        jax.ShapeDtypeStruct((NC, N, DH), jnp.float32),
        jax.ShapeDtypeStruct((N,), jnp.float32),
    ],
    mesh=plsc.VectorSubcoreMesh(core_axis_name="c", subcore_axis_name="s"),
    compiler_params=pltpu.CompilerParams(needs_layout_passes=False,
                                         use_tc_tiling_on_sc=False),
    scratch_types=(
        [
            pltpu.VMEM((NG, G), jnp.int32),
            pltpu.VMEM((NG, G), jnp.int32),
            pltpu.VMEM((N,), jnp.float32),
            pltpu.VMEM((N,), jnp.float32),
        ]
        + [pltpu.VMEM((G, DH), jnp.float32) for _ in range(UN)]
        + [pltpu.VMEM((G,), jnp.float32) for _ in range(UN)]
        + [
            pltpu.VMEM((160, DH), jnp.float32),
            pltpu.VMEM((640,), jnp.float32),
            pltpu.VMEM_SHARED((N, DH), jnp.float32),
            pltpu.VMEM_SHARED((N,), jnp.float32),
        ]
        + [pltpu.SemaphoreType.DMA for _ in range(UN + 2)]
    ),
)(_sc_body)


def _tc_norm_body(p_ref, d_ref, o_ref):
    num = jnp.concatenate([p_ref[0], p_ref[1]], axis=1)
    o_ref[...] = num / jnp.maximum(d_ref[...], 1e-16)


def _tc_norm(outp, denp):
    R = 2000
    return pl.pallas_call(
        _tc_norm_body,
        grid=(N // R,),
        in_specs=[
            pl.BlockSpec((NC, R, DH), lambda i: (0, i, 0)),
            pl.BlockSpec((R, 1), lambda i: (i, 0)),
        ],
        out_specs=pl.BlockSpec((R, D), lambda i: (i, 0)),
        out_shape=jax.ShapeDtypeStruct((N, D), jnp.float32),
    )(outp, denp)


def kernel(h, edge_index, ws_embed, W_fc, W_feat, b_feat, W_attn):
    src = edge_index[0].astype(jnp.int32).reshape(NS, NG, G)
    dst = edge_index[1].astype(jnp.int32).reshape(NS, NG, G)
    Wab = jnp.concatenate([W_attn[:D], W_attn[D:]], axis=1)
    z2, s2 = _tc_zs(h, W_fc, Wab)
    sa = s2[:, 0]
    sb = s2[:, 1]
    outp, denp = _sc_kernel(src, dst, sa, sb, z2)
    return _tc_norm(outp, denp.reshape(N, 1))
